# baseline (speedup 1.0000x reference)
import jax
import jax.numpy as jnp
from jax.experimental import pallas as pl
from jax.experimental.pallas import tpu as pltpu

_ROWS = 1000000
_DIM = 64
_BLK = 8000
_GRID = _ROWS // _BLK


def _body(idx_ref, emb_ref, new_ref, out_ref):
    i = pl.program_id(0)
    out_ref[...] = emb_ref[...]
    idx = idx_ref[0]

    @pl.when(idx // _BLK == i)
    def _():
        out_ref[pl.ds(idx - i * _BLK, 1), :] = new_ref[...]


def kernel(embeddings, emb_index, new_emb):
    idx = jnp.asarray(emb_index, jnp.int32).reshape(1)
    new_row = new_emb.reshape(1, _DIM)
    grid_spec = pltpu.PrefetchScalarGridSpec(
        num_scalar_prefetch=1,
        grid=(_GRID,),
        in_specs=[
            pl.BlockSpec((_BLK, _DIM), lambda i, idx_ref: (i, 0)),
            pl.BlockSpec((1, _DIM), lambda i, idx_ref: (0, 0)),
        ],
        out_specs=pl.BlockSpec((_BLK, _DIM), lambda i, idx_ref: (i, 0)),
    )
    return pl.pallas_call(
        _body,
        grid_spec=grid_spec,
        out_shape=jax.ShapeDtypeStruct((_ROWS, _DIM), embeddings.dtype),
    )(idx, embeddings, new_row)
